# baseline (speedup 1.0000x reference)
import functools

import jax
import jax.numpy as jnp
from jax import lax
from jax.experimental import pallas as pl
from jax.experimental.pallas import tpu as pltpu
from jax.experimental.pallas import tpu_sc as plsc

_B = 4096
_L = 200
_D = 64
_H = 256

_INFO = plsc.get_sparse_core_info()
_NC = _INFO.num_cores
_NS = _INFO.num_subcores
_NW = _NC * _NS
_RPW = _B // _NW
_C0 = 128
_C1 = _L - _C0
_UNROLL = 8


_NBUF = 2
_VP = 500001


def _pool_body(x_hbm, pairs_hbm, out_hbm, xpv, xv, rows0, rows1,
               outv, sem0, sem1):
    wid = lax.axis_index("s") * _NC + lax.axis_index("c")
    xbase = wid * _RPW * _L
    obase = wid * _RPW * _D
    bufs = (rows0, rows1)
    sems = (sem0, sem1)

    pltpu.sync_copy(x_hbm.at[pl.ds(xbase, _RPW * _L)], xv)

    def shift_body(r, carry):
        for k in range(13):
            o = r * _L + min(k * 16, _L - 16)
            xpv[pl.ds(o, 16)] = xv[pl.ds(o, 16)] >> 1
        return carry

    lax.fori_loop(0, _RPW, shift_body, 0)

    inv_l = jnp.full((16,), 1.0 / _L, dtype=jnp.float32)

    def _gather(r, buf, sem, issue):
        cp0 = pltpu.make_async_copy(
            pairs_hbm.at[xpv.at[pl.ds(r * _L, _C0)]], buf.at[pl.ds(0, _C0)],
            sem)
        cp1 = pltpu.make_async_copy(
            pairs_hbm.at[xpv.at[pl.ds(r * _L + _C0, _C1)]],
            buf.at[pl.ds(_C0, _C1)], sem)
        if issue:
            cp0.start()
            cp1.start()
        else:
            cp0.wait()
            cp1.wait()

    for r in range(_NBUF - 1):
        _gather(r, bufs[r], sems[r], issue=True)

    def iter_body(i, carry):
        for p in range(_NBUF):
            r = i * _NBUF + p
            nxt = r + (_NBUF - 1)

            @pl.when(nxt < _RPW)
            def _():
                _gather(nxt, bufs[(p + _NBUF - 1) % _NBUF],
                        sems[(p + _NBUF - 1) % _NBUF], issue=True)

            buf = bufs[p]
            _gather(r, buf, sems[p], issue=False)

            def acc16(jj0, pv, acc, lanes, buf=buf):
                a0, a1, a2, a3 = acc
                for u in lanes:
                    jj = jj0 + u
                    h = pv[u] * _D
                    a0 = a0 + buf[jj, pl.ds(h, 16)]
                    a1 = a1 + buf[jj, pl.ds(h + 16, 16)]
                    a2 = a2 + buf[jj, pl.ds(h + 32, 16)]
                    a3 = a3 + buf[jj, pl.ds(h + 48, 16)]
                return (a0, a1, a2, a3)

            def acc_body(j, acc):
                jj0 = j * 16
                pv = xv[pl.ds(r * _L + jj0, 16)] & 1
                return acc16(jj0, pv, acc, range(16))

            z = jnp.zeros((16,), dtype=jnp.float32)
            acc = lax.fori_loop(0, _L // 16, acc_body, (z, z, z, z))
            pv = xv[pl.ds(r * _L + _L - 16, 16)] & 1
            a0, a1, a2, a3 = acc16(_L - 16, pv, acc, range(8, 16))

            outv[pl.ds(r * _D, 16)] = a0 * inv_l
            outv[pl.ds(r * _D + 16, 16)] = a1 * inv_l
            outv[pl.ds(r * _D + 32, 16)] = a2 * inv_l
            outv[pl.ds(r * _D + 48, 16)] = a3 * inv_l
        return carry

    lax.fori_loop(0, _RPW // _NBUF, iter_body, 0)

    pltpu.sync_copy(outv, out_hbm.at[pl.ds(obase, _RPW * _D)])


@jax.jit
def _sc_pool(x, table):
    pairs = table.reshape(_VP, 2 * _D)
    mesh = plsc.VectorSubcoreMesh(core_axis_name="c", subcore_axis_name="s")
    out = pl.kernel(
        _pool_body,
        out_type=jax.ShapeDtypeStruct((_B * _D,), jnp.float32),
        mesh=mesh,
        scratch_types=[
            pltpu.VMEM((_RPW * _L,), jnp.int32),
            pltpu.VMEM((_RPW * _L,), jnp.int32),
            pltpu.VMEM((_L, 2 * _D), jnp.float32),
            pltpu.VMEM((_L, 2 * _D), jnp.float32),
            pltpu.VMEM((_RPW * _D,), jnp.float32),
            pltpu.SemaphoreType.DMA,
            pltpu.SemaphoreType.DMA,
        ],
    )(x.reshape(_B * _L), pairs)
    return out.reshape(_B, _D)


def _mlp_body(pooled_ref, w1_ref, b1_ref, w2_ref, b2_ref, out_ref):
    pooled = pooled_ref[...]
    hidden = lax.dot_general(
        pooled, w1_ref[...], (((1,), (1,)), ((), ())),
        preferred_element_type=jnp.float32)
    hidden = jnp.maximum(hidden + b1_ref[...], 0.0)
    out = jnp.sum(hidden * w2_ref[...], axis=1, keepdims=True)
    out_ref[...] = out + b2_ref[0]


@jax.jit
def _tc_mlp(pooled, W1, b1, W2, b2):
    out = pl.pallas_call(
        _mlp_body,
        in_specs=[
            pl.BlockSpec(memory_space=pltpu.VMEM),
            pl.BlockSpec(memory_space=pltpu.VMEM),
            pl.BlockSpec(memory_space=pltpu.VMEM),
            pl.BlockSpec(memory_space=pltpu.VMEM),
            pl.BlockSpec(memory_space=pltpu.SMEM),
        ],
        out_shape=jax.ShapeDtypeStruct((_B, 1), jnp.float32),
    )(pooled, W1, b1.reshape(1, _H), W2, b2)
    return jnp.squeeze(out, axis=-1)


def kernel(x, table, W1, b1, W2, b2):
    pooled = _sc_pool(x, table)
    return _tc_mlp(pooled, W1, b1, W2, b2)


# speedup vs baseline: 1.2065x; 1.2065x over previous
import functools

import jax
import jax.numpy as jnp
from jax import lax
from jax.experimental import pallas as pl
from jax.experimental.pallas import tpu as pltpu
from jax.experimental.pallas import tpu_sc as plsc

_B = 4096
_L = 200
_D = 64
_H = 256

_INFO = plsc.get_sparse_core_info()
_NC = _INFO.num_cores
_NS = _INFO.num_subcores
_NW = _NC * _NS
_RPW = _B // _NW
_C0 = 128
_C1 = _L - _C0
_UNROLL = 8


_NBUF = 2
_SPLIT = 500736
_CB = 1024
_NCB = _SPLIT // _CB
_ICB = 1000002 // _CB


def _pairs_body(tl_ref, tr_ref, out_ref):
    left = jnp.transpose(tl_ref[...])
    right = jnp.transpose(tr_ref[...])
    out_ref[...] = jnp.concatenate([left, right], axis=1)


@jax.jit
def _tc_pairs(table):
    t2 = jnp.swapaxes(table, 0, 1)
    return pl.pallas_call(
        _pairs_body,
        grid=(_NCB,),
        in_specs=[
            pl.BlockSpec((_D, _CB), lambda b: (0, b)),
            pl.BlockSpec((_D, _CB),
                         lambda b: (0, jnp.minimum(b + _NCB, _ICB))),
        ],
        out_specs=pl.BlockSpec((_CB, 2 * _D), lambda b: (b, 0)),
        out_shape=jax.ShapeDtypeStruct((_SPLIT, 2 * _D), jnp.float32),
    )(t2, t2)


def _pool_body(x_hbm, pairs_hbm, out_hbm, xpv, xv, rows0, rows1,
               outv, sem0, sem1):
    wid = lax.axis_index("s") * _NC + lax.axis_index("c")
    xbase = wid * _RPW * _L
    obase = wid * _RPW * _D
    bufs = (rows0, rows1)
    sems = (sem0, sem1)

    pltpu.sync_copy(x_hbm.at[pl.ds(xbase, _RPW * _L)], xv)

    def shift_body(r, carry):
        for k in range(13):
            o = r * _L + min(k * 16, _L - 16)
            c = xv[pl.ds(o, 16)]
            side = ((c - _SPLIT) >> 31) + 1
            xpv[pl.ds(o, 16)] = c - side * _SPLIT
        return carry

    lax.fori_loop(0, _RPW, shift_body, 0)

    inv_l = jnp.full((16,), 1.0 / _L, dtype=jnp.float32)

    def _gather(r, buf, sem, issue):
        cp0 = pltpu.make_async_copy(
            pairs_hbm.at[xpv.at[pl.ds(r * _L, _C0)]], buf.at[pl.ds(0, _C0)],
            sem)
        cp1 = pltpu.make_async_copy(
            pairs_hbm.at[xpv.at[pl.ds(r * _L + _C0, _C1)]],
            buf.at[pl.ds(_C0, _C1)], sem)
        if issue:
            cp0.start()
            cp1.start()
        else:
            cp0.wait()
            cp1.wait()

    for r in range(_NBUF - 1):
        _gather(r, bufs[r], sems[r], issue=True)

    def iter_body(i, carry):
        for p in range(_NBUF):
            r = i * _NBUF + p
            nxt = r + (_NBUF - 1)

            @pl.when(nxt < _RPW)
            def _():
                _gather(nxt, bufs[(p + _NBUF - 1) % _NBUF],
                        sems[(p + _NBUF - 1) % _NBUF], issue=True)

            buf = bufs[p]
            _gather(r, buf, sems[p], issue=False)

            def acc16(jj0, pv, acc, lanes, buf=buf):
                a0, a1, a2, a3 = acc
                for u in lanes:
                    jj = jj0 + u
                    h = pv[u] * _D
                    a0 = a0 + buf[jj, pl.ds(h, 16)]
                    a1 = a1 + buf[jj, pl.ds(h + 16, 16)]
                    a2 = a2 + buf[jj, pl.ds(h + 32, 16)]
                    a3 = a3 + buf[jj, pl.ds(h + 48, 16)]
                return (a0, a1, a2, a3)

            def acc_body(j, acc):
                jj0 = j * 16
                pv = ((xv[pl.ds(r * _L + jj0, 16)] - _SPLIT) >> 31) + 1
                return acc16(jj0, pv, acc, range(16))

            z = jnp.zeros((16,), dtype=jnp.float32)
            acc = lax.fori_loop(0, _L // 16, acc_body, (z, z, z, z))
            pv = ((xv[pl.ds(r * _L + _L - 16, 16)] - _SPLIT) >> 31) + 1
            a0, a1, a2, a3 = acc16(_L - 16, pv, acc, range(8, 16))

            outv[pl.ds(r * _D, 16)] = a0 * inv_l
            outv[pl.ds(r * _D + 16, 16)] = a1 * inv_l
            outv[pl.ds(r * _D + 32, 16)] = a2 * inv_l
            outv[pl.ds(r * _D + 48, 16)] = a3 * inv_l
        return carry

    lax.fori_loop(0, _RPW // _NBUF, iter_body, 0)

    pltpu.sync_copy(outv, out_hbm.at[pl.ds(obase, _RPW * _D)])


@jax.jit
def _sc_pool(x, pairs):
    mesh = plsc.VectorSubcoreMesh(core_axis_name="c", subcore_axis_name="s")
    out = pl.kernel(
        _pool_body,
        out_type=jax.ShapeDtypeStruct((_B * _D,), jnp.float32),
        mesh=mesh,
        scratch_types=[
            pltpu.VMEM((_RPW * _L,), jnp.int32),
            pltpu.VMEM((_RPW * _L,), jnp.int32),
            pltpu.VMEM((_L, 2 * _D), jnp.float32),
            pltpu.VMEM((_L, 2 * _D), jnp.float32),
            pltpu.VMEM((_RPW * _D,), jnp.float32),
            pltpu.SemaphoreType.DMA,
            pltpu.SemaphoreType.DMA,
        ],
    )(x.reshape(_B * _L), pairs)
    return out.reshape(_B, _D)


def _mlp_body(pooled_ref, w1_ref, b1_ref, w2_ref, b2_ref, out_ref):
    pooled = pooled_ref[...]
    hidden = lax.dot_general(
        pooled, w1_ref[...], (((1,), (1,)), ((), ())),
        preferred_element_type=jnp.float32)
    hidden = jnp.maximum(hidden + b1_ref[...], 0.0)
    out = jnp.sum(hidden * w2_ref[...], axis=1, keepdims=True)
    out_ref[...] = out + b2_ref[0]


@jax.jit
def _tc_mlp(pooled, W1, b1, W2, b2):
    out = pl.pallas_call(
        _mlp_body,
        in_specs=[
            pl.BlockSpec(memory_space=pltpu.VMEM),
            pl.BlockSpec(memory_space=pltpu.VMEM),
            pl.BlockSpec(memory_space=pltpu.VMEM),
            pl.BlockSpec(memory_space=pltpu.VMEM),
            pl.BlockSpec(memory_space=pltpu.SMEM),
        ],
        out_shape=jax.ShapeDtypeStruct((_B, 1), jnp.float32),
    )(pooled, W1, b1.reshape(1, _H), W2, b2)
    return jnp.squeeze(out, axis=-1)


def kernel(x, table, W1, b1, W2, b2):
    pairs = _tc_pairs(table)
    pooled = _sc_pool(x, pairs)
    return _tc_mlp(pooled, W1, b1, W2, b2)
